# baseline (speedup 1.0000x reference)
import functools

import jax
import jax.numpy as jnp
from jax import lax
from jax.experimental import pallas as pl
from jax.experimental.pallas import tpu as pltpu
from jax.experimental.pallas import tpu_sc as plsc

FD = 10000
BATCH = 1024
SCALE = 1.2
ROW = 3 * FD

NC, NS, L = 2, 16, 16
NW = NC * NS
RPW = BATCH // NW

TPAD = ((2 * FD + 127) // 128) * 128


def _copy_body(x_ref, o_ref):
    o_ref[...] = x_ref[...]


def _logits_copy(x):
    BB, CB = 256, 1280
    grid = (BATCH // BB, pl.cdiv(FD, CB))
    return pl.pallas_call(
        _copy_body,
        grid=grid,
        in_specs=[pl.BlockSpec((BB, CB), lambda i, j: (i, j))],
        out_specs=pl.BlockSpec((BB, CB), lambda i, j: (i, j)),
        out_shape=jax.ShapeDtypeStruct((BATCH, FD), jnp.float32),
    )(x)


def _sc_routed_build():
    mesh = plsc.VectorSubcoreMesh(core_axis_name="c", subcore_axis_name="s")

    @functools.partial(
        pl.kernel,
        out_type=(
            jax.ShapeDtypeStruct((BATCH, 128), jnp.float32),
            jax.ShapeDtypeStruct((BATCH, 128), jnp.float32),
            jax.ShapeDtypeStruct((BATCH, 128), jnp.float32),
            jax.ShapeDtypeStruct((BATCH,), jnp.int32),
            jax.ShapeDtypeStruct((BATCH,), jnp.int32),
        ),
        mesh=mesh,
        scratch_types=(
            pltpu.VMEM((RPW,), jnp.int32),
            pltpu.VMEM((RPW,), jnp.int32),
            pltpu.VMEM((RPW,), jnp.int32),
            pltpu.VMEM((RPW,), jnp.int32),
            pltpu.VMEM((RPW,), jnp.int32),
            pltpu.VMEM((RPW, 128), jnp.float32),
            pltpu.VMEM((RPW, 128), jnp.float32),
            pltpu.VMEM((RPW, 128), jnp.float32),
            pltpu.SemaphoreType.DMA,
            pltpu.SemaphoreType.DMA,
            pltpu.SemaphoreType.DMA,
        ),
    )
    def k(x128, labels, cc128, cs128,
          xchunk_o, cchunk_o, schunk_o, offx_o, offt_o,
          lab_v, xidx_v, tidx_v, offx_v, offt_v, xr_v, cr_v, sr_v,
          sem0, sem1, sem2):
        wid = lax.axis_index("s") * NC + lax.axis_index("c")
        base = wid * RPW
        pltpu.sync_copy(labels.at[pl.ds(base, RPW)], lab_v)

        iota = lax.iota(jnp.int32, L)
        for j in range(RPW // L):
            lab16 = lab_v[pl.ds(j * L, L)]
            el2 = lab16 * 2
            fullx = (base + j * L + iota) * ROW + FD + el2
            xidx_v[pl.ds(j * L, L)] = lax.shift_right_logical(fullx, 7)
            offx_v[pl.ds(j * L, L)] = jnp.bitwise_and(fullx, 127)
            tidx_v[pl.ds(j * L, L)] = lax.shift_right_logical(el2, 7)
            offt_v[pl.ds(j * L, L)] = jnp.bitwise_and(el2, 127)

        cpx = pltpu.async_copy(x128.at[xidx_v], xr_v, sem0)
        cpc = pltpu.async_copy(cc128.at[tidx_v], cr_v, sem1)
        cps = pltpu.async_copy(cs128.at[tidx_v], sr_v, sem2)
        cpx.wait()
        cpc.wait()
        cps.wait()

        pltpu.sync_copy(xr_v, xchunk_o.at[pl.ds(base, RPW)])
        pltpu.sync_copy(cr_v, cchunk_o.at[pl.ds(base, RPW)])
        pltpu.sync_copy(sr_v, schunk_o.at[pl.ds(base, RPW)])
        pltpu.sync_copy(offx_v, offx_o.at[pl.ds(base, RPW)])
        pltpu.sync_copy(offt_v, offt_o.at[pl.ds(base, RPW)])

    return k


def _extract_body(xc_ref, cc_ref, sc_ref, offx_ref, offt_ref,
                  gps_ref, size_ref, center_ref, reg_ref):
    lanes = lax.broadcasted_iota(jnp.int32, (BATCH, 128), 1)
    offx = offx_ref[...]
    offt = offt_ref[...]
    xc = xc_ref[...]
    cc = cc_ref[...]
    sc = sc_ref[...]

    def pick2(mat, off):
        a = jnp.sum(jnp.where(lanes == off, mat, 0.0), axis=1, keepdims=True)
        b = jnp.sum(jnp.where(lanes == off + 1, mat, 0.0), axis=1, keepdims=True)
        return a, b

    rlat, rlon = pick2(xc, offx)
    clat, clon = pick2(cc, offt)
    slat, slon = pick2(sc, offt)
    reg_lat = SCALE * jnp.tanh(rlat)
    reg_lon = SCALE * jnp.tanh(rlon)
    glat = jnp.clip(clat + reg_lat * slat * 0.5, -1.0, 1.0) * 90.0
    glon = jnp.clip(clon + reg_lon * slon * 0.5, -1.0, 1.0) * 180.0
    gps_ref[...] = jnp.concatenate([glat, glon], axis=1)
    size_ref[...] = jnp.concatenate([2.0 / slat, 2.0 / slon], axis=1)
    center_ref[...] = jnp.concatenate([clat, clon], axis=1)
    reg_ref[...] = jnp.concatenate([reg_lat, reg_lon], axis=1)


def _extract(xchunk, cchunk, schunk, offx, offt):
    o2 = jax.ShapeDtypeStruct((BATCH, 2), jnp.float32)
    return pl.pallas_call(
        _extract_body,
        out_shape=(o2, o2, o2, o2),
    )(xchunk, cchunk, schunk, offx, offt)


def kernel(x, gt_label, cell_center, cell_size):
    logits = _logits_copy(x)

    x128 = x.reshape(-1, 128)
    cc128 = jnp.pad(cell_center.reshape(-1), (0, TPAD - 2 * FD)).reshape(-1, 128)
    cs128 = jnp.pad(cell_size.reshape(-1), (0, TPAD - 2 * FD)).reshape(-1, 128)

    sc_k = _sc_routed_build()
    xchunk, cchunk, schunk, offx, offt = sc_k(
        x128, gt_label.astype(jnp.int32), cc128, cs128)

    gps, size, center, reg = _extract(
        xchunk, cchunk, schunk,
        offx.reshape(BATCH, 1), offt.reshape(BATCH, 1))
    return (logits, gps, size, center, reg)


# speedup vs baseline: 5.6611x; 5.6611x over previous
import functools

import jax
import jax.numpy as jnp
from jax import lax
from jax.experimental import pallas as pl
from jax.experimental.pallas import tpu as pltpu
from jax.experimental.pallas import tpu_sc as plsc

FD = 10000
BATCH = 1024
SCALE = 1.2

NC, NS, L = 2, 16, 16
NW = NC * NS
RPW = BATCH // NW

TPAD = ((2 * FD + 127) // 128) * 128


def _copy_body(x_ref, o_ref):
    o_ref[...] = x_ref[...]


def _logits_copy_t(xt):
    RB = 1000
    return pl.pallas_call(
        _copy_body,
        grid=(FD // RB,),
        in_specs=[pl.BlockSpec((RB, BATCH), lambda i: (i, 0))],
        out_specs=pl.BlockSpec((RB, BATCH), lambda i: (i, 0)),
        out_shape=jax.ShapeDtypeStruct((FD, BATCH), jnp.float32),
    )(xt)


def _sc_routed_build():
    mesh = plsc.VectorSubcoreMesh(core_axis_name="c", subcore_axis_name="s")

    @functools.partial(
        pl.kernel,
        out_type=(
            jax.ShapeDtypeStruct((2, BATCH), jnp.float32),
            jax.ShapeDtypeStruct((BATCH, 128), jnp.float32),
            jax.ShapeDtypeStruct((BATCH, 128), jnp.float32),
            jax.ShapeDtypeStruct((BATCH,), jnp.int32),
        ),
        mesh=mesh,
        scratch_types=(
            pltpu.VMEM((RPW,), jnp.int32),
            pltpu.VMEM((RPW,), jnp.int32),
            pltpu.VMEM((RPW,), jnp.int32),
            pltpu.VMEM((RPW,), jnp.int32),
            pltpu.VMEM((RPW,), jnp.int32),
            pltpu.VMEM((RPW, BATCH), jnp.float32),
            pltpu.VMEM((RPW, BATCH), jnp.float32),
            pltpu.VMEM((RPW, 128), jnp.float32),
            pltpu.VMEM((RPW, 128), jnp.float32),
            pltpu.VMEM((RPW,), jnp.float32),
            pltpu.VMEM((RPW,), jnp.float32),
            pltpu.SemaphoreType.DMA,
            pltpu.SemaphoreType.DMA,
            pltpu.SemaphoreType.DMA,
            pltpu.SemaphoreType.DMA,
        ),
    )
    def k(xt, labels, cc128, cs128,
          reg_o, cchunk_o, schunk_o, offt_o,
          lab_v, ilat_v, ilon_v, tidx_v, offt_v,
          glat_v, glon_v, cr_v, sr_v, rlat_v, rlon_v,
          sem0, sem1, sem2, sem3):
        wid = lax.axis_index("s") * NC + lax.axis_index("c")
        base = wid * RPW
        pltpu.sync_copy(labels.at[pl.ds(base, RPW)], lab_v)

        iota = lax.iota(jnp.int32, L)
        for j in range(RPW // L):
            lab16 = lab_v[pl.ds(j * L, L)]
            el2 = lab16 * 2
            ilat_v[pl.ds(j * L, L)] = FD + el2
            ilon_v[pl.ds(j * L, L)] = FD + el2 + 1
            tidx_v[pl.ds(j * L, L)] = lax.shift_right_logical(el2, 7)
            offt_v[pl.ds(j * L, L)] = jnp.bitwise_and(el2, 127)

        cp0 = pltpu.async_copy(xt.at[ilat_v], glat_v, sem0)
        cp1 = pltpu.async_copy(xt.at[ilon_v], glon_v, sem1)
        cp2 = pltpu.async_copy(cc128.at[tidx_v], cr_v, sem2)
        cp3 = pltpu.async_copy(cs128.at[tidx_v], sr_v, sem3)
        cp0.wait()
        cp1.wait()
        cp2.wait()
        cp3.wait()

        for gv, rv in ((glat_v, rlat_v), (glon_v, rlon_v)):
            for j in range(RPW // L):
                acc = jnp.zeros((L,), jnp.float32)
                for i in range(L):
                    r = j * L + i
                    win = base + (r // L) * L
                    v16 = gv[r, pl.ds(win, L)]
                    acc = jnp.where(iota == i, v16[r % L], acc)
                e = jnp.exp(acc * 2.0)
                acc = SCALE * (1.0 - 2.0 / (e + 1.0))
                rv[pl.ds(j * L, L)] = acc

        pltpu.sync_copy(rlat_v, reg_o.at[0, pl.ds(base, RPW)])
        pltpu.sync_copy(rlon_v, reg_o.at[1, pl.ds(base, RPW)])
        pltpu.sync_copy(cr_v, cchunk_o.at[pl.ds(base, RPW)])
        pltpu.sync_copy(sr_v, schunk_o.at[pl.ds(base, RPW)])
        pltpu.sync_copy(offt_v, offt_o.at[pl.ds(base, RPW)])

    return k


def _finish_body(cc_ref, sc_ref, offt_ref, reg_ref,
                 gps_ref, size_ref, center_ref):
    lanes = lax.broadcasted_iota(jnp.int32, (BATCH, 128), 1)
    offt = offt_ref[...]
    cc = cc_ref[...]
    sc = sc_ref[...]

    def pick2(mat):
        a = jnp.sum(jnp.where(lanes == offt, mat, 0.0), axis=1, keepdims=True)
        b = jnp.sum(jnp.where(lanes == offt + 1, mat, 0.0), axis=1, keepdims=True)
        return a, b

    clat, clon = pick2(cc)
    slat, slon = pick2(sc)
    reg = reg_ref[...]
    reg_lat = reg[:, 0:1]
    reg_lon = reg[:, 1:2]
    glat = jnp.clip(clat + reg_lat * slat * 0.5, -1.0, 1.0) * 90.0
    glon = jnp.clip(clon + reg_lon * slon * 0.5, -1.0, 1.0) * 180.0
    gps_ref[...] = jnp.concatenate([glat, glon], axis=1)
    size_ref[...] = jnp.concatenate([2.0 / slat, 2.0 / slon], axis=1)
    center_ref[...] = jnp.concatenate([clat, clon], axis=1)


def _finish(cchunk, schunk, offt, reg):
    o2 = jax.ShapeDtypeStruct((BATCH, 2), jnp.float32)
    return pl.pallas_call(
        _finish_body,
        out_shape=(o2, o2, o2),
    )(cchunk, schunk, offt, reg)


def kernel(x, gt_label, cell_center, cell_size):
    xt = x.T
    logits = _logits_copy_t(xt).T

    cc128 = jnp.pad(cell_center.reshape(-1), (0, TPAD - 2 * FD)).reshape(-1, 128)
    cs128 = jnp.pad(cell_size.reshape(-1), (0, TPAD - 2 * FD)).reshape(-1, 128)

    sc_k = _sc_routed_build()
    reg_t, cchunk, schunk, offt = sc_k(
        xt, gt_label.astype(jnp.int32), cc128, cs128)
    reg = reg_t.T

    gps, size, center = _finish(cchunk, schunk, offt.reshape(BATCH, 1), reg)
    return (logits, gps, size, center, reg)


# speedup vs baseline: 7.4669x; 1.3190x over previous
import functools

import jax
import jax.numpy as jnp
from jax import lax
from jax.experimental import pallas as pl
from jax.experimental.pallas import tpu as pltpu
from jax.experimental.pallas import tpu_sc as plsc

FD = 10000
BATCH = 1024
SCALE = 1.2

NC, NS, L = 2, 16, 16
NW = NC * NS
RPW = BATCH // NW


def _copy_body(x_ref, o_ref):
    o_ref[...] = x_ref[...]


def _logits_copy_t(xt):
    RB = 1000
    return pl.pallas_call(
        _copy_body,
        grid=(FD // RB,),
        in_specs=[pl.BlockSpec((RB, BATCH), lambda i: (i, 0))],
        out_specs=pl.BlockSpec((RB, BATCH), lambda i: (i, 0)),
        out_shape=jax.ShapeDtypeStruct((FD, BATCH), jnp.float32),
    )(xt)


def _sc_routed_build():
    mesh = plsc.VectorSubcoreMesh(core_axis_name="c", subcore_axis_name="s")

    o2 = jax.ShapeDtypeStruct((2, BATCH), jnp.float32)

    @functools.partial(
        pl.kernel,
        out_type=(o2, o2, o2, o2),
        mesh=mesh,
        scratch_types=(
            pltpu.VMEM((RPW,), jnp.int32),
            pltpu.VMEM((RPW,), jnp.int32),
            pltpu.VMEM((RPW,), jnp.int32),
            pltpu.VMEM((RPW, 128), jnp.float32),
            pltpu.VMEM((RPW, 128), jnp.float32),
            pltpu.VMEM((RPW, 128), jnp.float32),
            pltpu.VMEM((RPW, 128), jnp.float32),
            pltpu.VMEM((8, RPW), jnp.float32),
            pltpu.SemaphoreType.DMA,
            pltpu.SemaphoreType.DMA,
            pltpu.SemaphoreType.DMA,
            pltpu.SemaphoreType.DMA,
        ),
    )
    def k(xt, labels, ccp, csp,
          gps_o, size_o, center_o, reg_o,
          lab_v, ilat_v, ilon_v, glat_v, glon_v, gc_v, gs_v, st_v,
          sem0, sem1, sem2, sem3):
        wid = lax.axis_index("s") * NC + lax.axis_index("c")
        base = wid * RPW
        pltpu.sync_copy(labels.at[pl.ds(base, RPW)], lab_v)

        iota = lax.iota(jnp.int32, L)
        zero16 = iota.astype(jnp.float32) * 0.0
        for j in range(RPW // L):
            lab16 = lab_v[pl.ds(j * L, L)]
            el2 = lab16 * 2
            ilat_v[pl.ds(j * L, L)] = FD + el2
            ilon_v[pl.ds(j * L, L)] = FD + el2 + 1

        win = lax.shift_right_logical(base, 7) * 128
        cp0 = pltpu.async_copy(xt.at[ilat_v, pl.ds(win, 128)], glat_v, sem0)
        cp1 = pltpu.async_copy(xt.at[ilon_v, pl.ds(win, 128)], glon_v, sem1)
        cp2 = pltpu.async_copy(ccp.at[lab_v], gc_v, sem2)
        cp3 = pltpu.async_copy(csp.at[lab_v], gs_v, sem3)
        cp0.wait()
        cp1.wait()
        cp2.wait()
        cp3.wait()

        woff = jnp.bitwise_and(base, 127)
        for j in range(RPW // L):
            rlat = zero16
            rlon = zero16
            clat = zero16
            clon = zero16
            slat = zero16
            slon = zero16
            for i in range(L):
                r = j * L + i
                w = woff + (r // L) * L
                pick = iota == i
                rlat = jnp.where(pick, glat_v[r, pl.ds(w, L)][r % L], rlat)
                rlon = jnp.where(pick, glon_v[r, pl.ds(w, L)][r % L], rlon)
                vc = gc_v[r, pl.ds(0, L)]
                clat = jnp.where(pick, vc[0], clat)
                clon = jnp.where(pick, vc[1], clon)
                vs = gs_v[r, pl.ds(0, L)]
                slat = jnp.where(pick, vs[0], slat)
                slon = jnp.where(pick, vs[1], slon)

            elat = jnp.exp(rlat * 2.0)
            rlat = SCALE * (1.0 - 2.0 / (elat + 1.0))
            elon = jnp.exp(rlon * 2.0)
            rlon = SCALE * (1.0 - 2.0 / (elon + 1.0))
            glat = jnp.clip(clat + rlat * slat * 0.5, -1.0, 1.0) * 90.0
            glon = jnp.clip(clon + rlon * slon * 0.5, -1.0, 1.0) * 180.0

            sl = pl.ds(j * L, L)
            st_v[0, sl] = glat
            st_v[1, sl] = glon
            st_v[2, sl] = 2.0 / slat
            st_v[3, sl] = 2.0 / slon
            st_v[4, sl] = clat
            st_v[5, sl] = clon
            st_v[6, sl] = rlat
            st_v[7, sl] = rlon

        bsl = pl.ds(base, RPW)
        pltpu.sync_copy(st_v.at[0], gps_o.at[0, bsl])
        pltpu.sync_copy(st_v.at[1], gps_o.at[1, bsl])
        pltpu.sync_copy(st_v.at[2], size_o.at[0, bsl])
        pltpu.sync_copy(st_v.at[3], size_o.at[1, bsl])
        pltpu.sync_copy(st_v.at[4], center_o.at[0, bsl])
        pltpu.sync_copy(st_v.at[5], center_o.at[1, bsl])
        pltpu.sync_copy(st_v.at[6], reg_o.at[0, bsl])
        pltpu.sync_copy(st_v.at[7], reg_o.at[1, bsl])

    return k


def kernel(x, gt_label, cell_center, cell_size):
    xt = x.T
    logits = _logits_copy_t(xt).T

    ccp = jnp.pad(cell_center, ((0, 0), (0, 126)))
    csp = jnp.pad(cell_size, ((0, 0), (0, 126)))

    sc_k = _sc_routed_build()
    gps_t, size_t, center_t, reg_t = sc_k(
        xt, gt_label.astype(jnp.int32), ccp, csp)

    return (logits, gps_t.T, size_t.T, center_t.T, reg_t.T)
